# baseline (speedup 1.0000x reference)
import functools

import jax
import jax.numpy as jnp
import numpy as np
from jax.experimental import pallas as pl
from jax.experimental.pallas import tpu as pltpu

_N_SCALAR = 32
_N_VEC = 16
_N_TEN = 8
_DIM = _N_SCALAR + 3 * _N_VEC + 5 * _N_TEN
_NG = _N_VEC + _N_TEN
_EPS = 1e-05
_BLK = 4096


def _selectors():
    v = np.zeros((_DIM, 128), np.float32)
    for g in range(_N_VEC):
        v[_N_SCALAR + 3 * g:_N_SCALAR + 3 * (g + 1), g] = 1.0
    for t in range(_N_TEN):
        base = _N_SCALAR + 3 * _N_VEC + 5 * t
        v[base:base + 5, _N_VEC + t] = 1.0
    return jnp.asarray(v), jnp.asarray(v.T.copy())


def _p1_body(x_ref, v_ref, o_ref, *, n_rows):
    i = pl.program_id(0)
    x = x_ref[...]
    norm2 = jax.lax.dot(x * x, v_ref[...],
                        precision=jax.lax.Precision.HIGHEST,
                        preferred_element_type=jnp.float32)
    norm = jnp.maximum(jnp.sqrt(norm2), 1e-06)
    rows = jax.lax.broadcasted_iota(jnp.int32, norm.shape, 0) + i * _BLK
    norm = jnp.where(rows < n_rows, norm, 0.0)
    psum = jnp.sum(norm, axis=0, keepdims=True)

    @pl.when(i == 0)
    def _():
        o_ref[...] = jnp.zeros_like(o_ref)

    o_ref[...] += psum


def _p2_body(x_ref, v_ref, g_ref, sums_ref, wpad_ref, bpad_ref, o_ref, *,
             n_rows):
    x = x_ref[...]
    scal = x[:, :_N_SCALAR]
    mu = jnp.sum(scal, axis=1, keepdims=True) * (1.0 / _N_SCALAR)
    ex2 = jnp.sum(scal * scal, axis=1, keepdims=True) * (1.0 / _N_SCALAR)
    r = jax.lax.rsqrt(jnp.maximum(ex2 - mu * mu, 0.0) + _EPS)
    norm2 = jax.lax.dot(x * x, v_ref[...],
                        precision=jax.lax.Precision.HIGHEST,
                        preferred_element_type=jnp.float32)
    norm = jnp.maximum(jnp.sqrt(norm2), 1e-06)
    s = sums_ref[...] * (1.0 / n_rows) / norm
    scale_vt = jax.lax.dot(s, g_ref[...],
                           precision=jax.lax.Precision.HIGHEST,
                           preferred_element_type=jnp.float32)
    wpad = wpad_ref[...]
    a = r * wpad + scale_vt
    b = bpad_ref[...] - (mu * r) * wpad
    o_ref[...] = x * a + b


def kernel(x, weight, bias):
    n = x.shape[0]
    grid = (pl.cdiv(n, _BLK),)
    v, g = _selectors()
    wpad = jnp.zeros((1, _DIM), jnp.float32).at[0, :_N_SCALAR].set(weight)
    bpad = jnp.zeros((1, _DIM), jnp.float32).at[0, :_N_SCALAR].set(bias)

    sums = pl.pallas_call(
        functools.partial(_p1_body, n_rows=n),
        grid=grid,
        in_specs=[
            pl.BlockSpec((_BLK, _DIM), lambda i: (i, 0)),
            pl.BlockSpec((_DIM, 128), lambda i: (0, 0)),
        ],
        out_specs=pl.BlockSpec((1, 128), lambda i: (0, 0)),
        out_shape=jax.ShapeDtypeStruct((1, 128), jnp.float32),
        compiler_params=pltpu.CompilerParams(
            dimension_semantics=("arbitrary",)),
    )(x, v)

    out = pl.pallas_call(
        functools.partial(_p2_body, n_rows=n),
        grid=grid,
        in_specs=[
            pl.BlockSpec((_BLK, _DIM), lambda i: (i, 0)),
            pl.BlockSpec((_DIM, 128), lambda i: (0, 0)),
            pl.BlockSpec((128, _DIM), lambda i: (0, 0)),
            pl.BlockSpec((1, 128), lambda i: (0, 0)),
            pl.BlockSpec((1, _DIM), lambda i: (0, 0)),
            pl.BlockSpec((1, _DIM), lambda i: (0, 0)),
        ],
        out_specs=pl.BlockSpec((_BLK, _DIM), lambda i: (i, 0)),
        out_shape=jax.ShapeDtypeStruct((n, _DIM), jnp.float32),
        compiler_params=pltpu.CompilerParams(
            dimension_semantics=("arbitrary",)),
    )(x, v, g, sums, wpad, bpad)
    return out


# speedup vs baseline: 1.9171x; 1.9171x over previous
import functools

import jax
import jax.numpy as jnp
import numpy as np
from jax.experimental import pallas as pl
from jax.experimental.pallas import tpu as pltpu

_N_SCALAR = 32
_N_VEC = 16
_N_TEN = 8
_DIM = _N_SCALAR + 3 * _N_VEC + 5 * _N_TEN
_NG = _N_VEC + _N_TEN
_EPS = 1e-05
_BLK = 4096
_PREC = jax.lax.Precision.DEFAULT


def _group_cols():
    pairs = []
    for g in range(_N_VEC):
        for k in range(3):
            pairs.append((_N_SCALAR + 3 * g + k, g))
    for t in range(_N_TEN):
        for k in range(5):
            pairs.append((_N_SCALAR + 3 * _N_VEC + 5 * t + k, _N_VEC + t))
    return pairs


def _const_mats():
    vz = np.zeros((_DIM, 128), np.float32)
    for c, g in _group_cols():
        vz[c, g] = 1.0
    vz[:_N_SCALAR, 24] = 1.0 / _N_SCALAR
    vz[:_N_SCALAR, 25] = 1.0 / _N_SCALAR
    vx = np.zeros((_DIM, 128), np.float32)
    vx[:_N_SCALAR, 24] = 1.0 / _N_SCALAR
    vx[:_N_SCALAR, 25] = 1.0 / _N_SCALAR
    ev = np.zeros((1, 128), np.float32)
    ev[0, 24] = _EPS
    ev[0, 25] = _EPS
    m25 = np.zeros((1, 128), np.float32)
    m25[0, 25] = 1.0
    m26 = np.zeros((1, 128), np.float32)
    m26[0, 26] = 1.0
    return (jnp.asarray(vz), jnp.asarray(vx), jnp.asarray(ev),
            jnp.asarray(m25), jnp.asarray(m26))


def _p1_body(x_ref, vz_ref, o_ref, *, n_rows):
    i = pl.program_id(0)
    x = x_ref[...]
    norm2 = jax.lax.dot(x * x, vz_ref[...], precision=_PREC,
                        preferred_element_type=jnp.float32)
    n2c = jnp.maximum(norm2, 1e-12)
    norm = jnp.maximum(norm2 * jax.lax.rsqrt(n2c), 1e-06)
    rows = jax.lax.broadcasted_iota(jnp.int32, norm.shape, 0) + i * _BLK
    norm = jnp.where(rows < n_rows, norm, 0.0)
    psum = jnp.sum(norm, axis=0, keepdims=True)

    @pl.when(i == 0)
    def _():
        o_ref[...] = jnp.zeros_like(o_ref)

    o_ref[...] += psum


def _p2_body(x_ref, vz_ref, vx_ref, g_ref, sums_ref, ev_ref, m25_ref,
             m26_ref, o_ref, *, n_rows):
    x = x_ref[...]
    mz = jax.lax.dot(x * x, vz_ref[...], precision=_PREC,
                     preferred_element_type=jnp.float32)
    mx = jax.lax.dot(x, vx_ref[...], precision=_PREC,
                     preferred_element_type=jnp.float32)
    t = jnp.maximum(mz - mx * mx + ev_ref[...], 1e-12)
    rall = jax.lax.rsqrt(t)
    col = jax.lax.broadcasted_iota(jnp.int32, (1, 128), 1)
    coef = jnp.where(col < _NG, sums_ref[...] * (1.0 / n_rows),
                     jnp.where(col == _NG, 1.0, 0.0))
    s = rall * coef + (mx * rall) * m25_ref[...] + m26_ref[...]
    ab = jax.lax.dot(s, g_ref[...], precision=_PREC,
                     preferred_element_type=jnp.float32)
    o_ref[...] = x * ab[:, :_DIM] + ab[:, 128:128 + _DIM]


def kernel(x, weight, bias):
    n = x.shape[0]
    grid = (pl.cdiv(n, _BLK),)
    vz, vx, ev, m25, m26 = _const_mats()

    ga = np.zeros((128, 128), np.float32)
    for c, g in _group_cols():
        ga[g, c] = 1.0
    ga = jnp.asarray(ga).at[24, :_N_SCALAR].set(weight)
    gb = jnp.zeros((128, 128), jnp.float32)
    gb = gb.at[25, :_N_SCALAR].set(-weight)
    gb = gb.at[26, :_N_SCALAR].set(bias)
    g = jnp.concatenate([ga, gb], axis=1)

    sums = pl.pallas_call(
        functools.partial(_p1_body, n_rows=n),
        grid=grid,
        in_specs=[
            pl.BlockSpec((_BLK, _DIM), lambda i: (i, 0)),
            pl.BlockSpec((_DIM, 128), lambda i: (0, 0)),
        ],
        out_specs=pl.BlockSpec((1, 128), lambda i: (0, 0)),
        out_shape=jax.ShapeDtypeStruct((1, 128), jnp.float32),
        compiler_params=pltpu.CompilerParams(
            dimension_semantics=("arbitrary",)),
    )(x, vz)

    out = pl.pallas_call(
        functools.partial(_p2_body, n_rows=n),
        grid=grid,
        in_specs=[
            pl.BlockSpec((_BLK, _DIM), lambda i: (i, 0)),
            pl.BlockSpec((_DIM, 128), lambda i: (0, 0)),
            pl.BlockSpec((_DIM, 128), lambda i: (0, 0)),
            pl.BlockSpec((128, 256), lambda i: (0, 0)),
            pl.BlockSpec((1, 128), lambda i: (0, 0)),
            pl.BlockSpec((1, 128), lambda i: (0, 0)),
            pl.BlockSpec((1, 128), lambda i: (0, 0)),
            pl.BlockSpec((1, 128), lambda i: (0, 0)),
        ],
        out_specs=pl.BlockSpec((_BLK, _DIM), lambda i: (i, 0)),
        out_shape=jax.ShapeDtypeStruct((n, _DIM), jnp.float32),
        compiler_params=pltpu.CompilerParams(
            dimension_semantics=("arbitrary",)),
    )(x, vz, vx, g, sums, ev, m25, m26)
    return out


# speedup vs baseline: 1.9506x; 1.0175x over previous
import functools

import jax
import jax.numpy as jnp
import numpy as np
from jax.experimental import pallas as pl
from jax.experimental.pallas import tpu as pltpu

_N_SCALAR = 32
_N_VEC = 16
_N_TEN = 8
_DIM = _N_SCALAR + 3 * _N_VEC + 5 * _N_TEN
_NG = _N_VEC + _N_TEN
_EPS = 1e-05
_BLK = 4096
_PREC = jax.lax.Precision.DEFAULT


def _group_cols():
    pairs = []
    for g in range(_N_VEC):
        for k in range(3):
            pairs.append((_N_SCALAR + 3 * g + k, g))
    for t in range(_N_TEN):
        for k in range(5):
            pairs.append((_N_SCALAR + 3 * _N_VEC + 5 * t + k, _N_VEC + t))
    return pairs


def _const_mats():
    vz = np.zeros((_DIM, 128), np.float32)
    for c, g in _group_cols():
        vz[c, g] = 1.0
    vz[:_N_SCALAR, 24] = 1.0 / _N_SCALAR
    vz[:_N_SCALAR, 25] = 1.0 / _N_SCALAR
    vx = np.zeros((_DIM, 128), np.float32)
    vx[:_N_SCALAR, 24] = 1.0 / _N_SCALAR
    vx[:_N_SCALAR, 25] = 1.0 / _N_SCALAR
    ev = np.zeros((1, 128), np.float32)
    ev[0, 24] = _EPS
    ev[0, 25] = _EPS
    m25 = np.zeros((1, 128), np.float32)
    m25[0, 25] = 1.0
    m26 = np.zeros((1, 128), np.float32)
    m26[0, 26] = 1.0
    gc = np.zeros((128, 256), np.float32)
    for c, g in _group_cols():
        gc[g, c] = 1.0
    return (jnp.asarray(vz), jnp.asarray(vx), jnp.asarray(ev),
            jnp.asarray(m25), jnp.asarray(m26), jnp.asarray(gc))


def _p1_body(x_ref, vz_ref, o_ref, *, n_rows):
    i = pl.program_id(0)
    x = x_ref[...]
    norm2 = jax.lax.dot(x * x, vz_ref[...], precision=_PREC,
                        preferred_element_type=jnp.float32)
    n2c = jnp.maximum(norm2, 1e-12)
    norm = jnp.maximum(norm2 * jax.lax.rsqrt(n2c), 1e-06)
    rows = jax.lax.broadcasted_iota(jnp.int32, norm.shape, 0) + i * _BLK
    norm = jnp.where(rows < n_rows, norm, 0.0)
    psum = jnp.sum(norm, axis=0, keepdims=True)

    @pl.when(i == 0)
    def _():
        o_ref[...] = jnp.zeros_like(o_ref)

    o_ref[...] += psum


def _p2_body(x_ref, w_ref, b_ref, vz_ref, vx_ref, gc_ref, sums_ref, ev_ref,
             m25_ref, m26_ref, o_ref, *, n_rows):
    x = x_ref[...]
    mz = jax.lax.dot(x * x, vz_ref[...], precision=_PREC,
                     preferred_element_type=jnp.float32)
    mx = jax.lax.dot(x, vx_ref[...], precision=_PREC,
                     preferred_element_type=jnp.float32)
    t = jnp.maximum(mz - mx * mx + ev_ref[...], 1e-12)
    rall = jax.lax.rsqrt(t)
    col1 = jax.lax.broadcasted_iota(jnp.int32, (1, 128), 1)
    coef = jnp.where(col1 < _NG, sums_ref[...] * (1.0 / n_rows),
                     jnp.where(col1 == _NG, 1.0, 0.0))
    s = rall * coef + (mx * rall) * m25_ref[...] + m26_ref[...]
    row = jax.lax.broadcasted_iota(jnp.int32, (128, 256), 0)
    col = jax.lax.broadcasted_iota(jnp.int32, (128, 256), 1)
    zpad = jnp.zeros((1, 128 - _N_SCALAR), jnp.float32)
    w = jnp.concatenate([w_ref[...], zpad], axis=1)
    b = jnp.concatenate([b_ref[...], zpad], axis=1)
    wtile = jnp.concatenate([w, w], axis=1)
    btile = jnp.concatenate([b, b], axis=1)
    in_a = col < 128
    g = gc_ref[...]
    g = jnp.where((row == 24) & in_a, wtile, g)
    g = jnp.where((row == 25) & ~in_a, -wtile, g)
    g = jnp.where((row == 26) & ~in_a, btile, g)
    ab = jax.lax.dot(s, g, precision=_PREC,
                     preferred_element_type=jnp.float32)
    o_ref[...] = x * ab[:, :_DIM] + ab[:, 128:128 + _DIM]


def kernel(x, weight, bias):
    n = x.shape[0]
    grid = (pl.cdiv(n, _BLK),)
    vz, vx, ev, m25, m26, gc = _const_mats()
    w2 = weight.reshape(1, _N_SCALAR)
    b2 = bias.reshape(1, _N_SCALAR)

    sums = pl.pallas_call(
        functools.partial(_p1_body, n_rows=n),
        grid=grid,
        in_specs=[
            pl.BlockSpec((_BLK, _DIM), lambda i: (i, 0)),
            pl.BlockSpec((_DIM, 128), lambda i: (0, 0)),
        ],
        out_specs=pl.BlockSpec((1, 128), lambda i: (0, 0)),
        out_shape=jax.ShapeDtypeStruct((1, 128), jnp.float32),
        compiler_params=pltpu.CompilerParams(
            dimension_semantics=("arbitrary",)),
    )(x, vz)

    out = pl.pallas_call(
        functools.partial(_p2_body, n_rows=n),
        grid=grid,
        in_specs=[
            pl.BlockSpec((_BLK, _DIM), lambda i: (i, 0)),
            pl.BlockSpec((1, _N_SCALAR), lambda i: (0, 0)),
            pl.BlockSpec((1, _N_SCALAR), lambda i: (0, 0)),
            pl.BlockSpec((_DIM, 128), lambda i: (0, 0)),
            pl.BlockSpec((_DIM, 128), lambda i: (0, 0)),
            pl.BlockSpec((128, 256), lambda i: (0, 0)),
            pl.BlockSpec((1, 128), lambda i: (0, 0)),
            pl.BlockSpec((1, 128), lambda i: (0, 0)),
            pl.BlockSpec((1, 128), lambda i: (0, 0)),
            pl.BlockSpec((1, 128), lambda i: (0, 0)),
        ],
        out_specs=pl.BlockSpec((_BLK, _DIM), lambda i: (i, 0)),
        out_shape=jax.ShapeDtypeStruct((n, _DIM), jnp.float32),
        compiler_params=pltpu.CompilerParams(
            dimension_semantics=("arbitrary",)),
    )(x, w2, b2, vz, vx, gc, sums, ev, m25, m26)
    return out


# speedup vs baseline: 2.0810x; 1.0668x over previous
import functools

import jax
import jax.numpy as jnp
import numpy as np
from jax.experimental import pallas as pl
from jax.experimental.pallas import tpu as pltpu

_N_SCALAR = 32
_N_VEC = 16
_N_TEN = 8
_DIM = _N_SCALAR + 3 * _N_VEC + 5 * _N_TEN
_NG = _N_VEC + _N_TEN
_EPS = 1e-05
_BLK = 8192
_PREC = jax.lax.Precision.DEFAULT


def _group_cols():
    pairs = []
    for g in range(_N_VEC):
        for k in range(3):
            pairs.append((_N_SCALAR + 3 * g + k, g))
    for t in range(_N_TEN):
        for k in range(5):
            pairs.append((_N_SCALAR + 3 * _N_VEC + 5 * t + k, _N_VEC + t))
    return pairs


def _const_mats():
    vz = np.zeros((_DIM, 128), np.float32)
    for c, g in _group_cols():
        vz[c, g] = 1.0
    vz[:_N_SCALAR, 24] = 1.0 / _N_SCALAR
    vz[:_N_SCALAR, 25] = 1.0 / _N_SCALAR
    vx = np.zeros((_DIM, 128), np.float32)
    vx[:_N_SCALAR, 24] = 1.0 / _N_SCALAR
    vx[:_N_SCALAR, 25] = 1.0 / _N_SCALAR
    ev = np.zeros((1, 128), np.float32)
    ev[0, 24] = _EPS
    ev[0, 25] = _EPS
    m25 = np.zeros((1, 128), np.float32)
    m25[0, 25] = 1.0
    m26 = np.zeros((1, 128), np.float32)
    m26[0, 26] = 1.0
    gc = np.zeros((128, 256), np.float32)
    for c, g in _group_cols():
        gc[g, c] = 1.0
    return (jnp.asarray(vz), jnp.asarray(vx), jnp.asarray(ev),
            jnp.asarray(m25), jnp.asarray(m26), jnp.asarray(gc))


def _body(x_ref, w_ref, b_ref, vz_ref, vx_ref, gc_ref, ev_ref, m25_ref,
          m26_ref, o_ref, acc_ref, *, n_rows):
    p = pl.program_id(0)
    i = pl.program_id(1)
    x = x_ref[...]

    @pl.when(p == 0)
    def _phase0():
        norm2 = jax.lax.dot(x * x, vz_ref[...], precision=_PREC,
                            preferred_element_type=jnp.float32)
        n2c = jnp.maximum(norm2, 1e-12)
        norm = jnp.maximum(norm2 * jax.lax.rsqrt(n2c), 1e-06)
        rows = jax.lax.broadcasted_iota(jnp.int32, norm.shape, 0) + i * _BLK
        norm = jnp.where(rows < n_rows, norm, 0.0)
        psum = jnp.sum(norm, axis=0, keepdims=True)

        @pl.when(i == 0)
        def _():
            acc_ref[...] = jnp.zeros_like(acc_ref)

        acc_ref[...] += psum

    @pl.when(p == 1)
    def _phase1():
        mz = jax.lax.dot(x * x, vz_ref[...], precision=_PREC,
                         preferred_element_type=jnp.float32)
        mx = jax.lax.dot(x, vx_ref[...], precision=_PREC,
                         preferred_element_type=jnp.float32)
        t = jnp.maximum(mz - mx * mx + ev_ref[...], 1e-12)
        rall = jax.lax.rsqrt(t)
        col1 = jax.lax.broadcasted_iota(jnp.int32, (1, 128), 1)
        coef = jnp.where(col1 < _NG, acc_ref[...] * (1.0 / n_rows),
                         jnp.where(col1 == _NG, 1.0, 0.0))
        s = rall * coef + (mx * rall) * m25_ref[...] + m26_ref[...]
        row = jax.lax.broadcasted_iota(jnp.int32, (128, 256), 0)
        col = jax.lax.broadcasted_iota(jnp.int32, (128, 256), 1)
        zpad = jnp.zeros((1, 128 - _N_SCALAR), jnp.float32)
        w = jnp.concatenate([w_ref[...], zpad], axis=1)
        b = jnp.concatenate([b_ref[...], zpad], axis=1)
        wtile = jnp.concatenate([w, w], axis=1)
        btile = jnp.concatenate([b, b], axis=1)
        in_a = col < 128
        g = gc_ref[...]
        g = jnp.where((row == 24) & in_a, wtile, g)
        g = jnp.where((row == 25) & ~in_a, -wtile, g)
        g = jnp.where((row == 26) & ~in_a, btile, g)
        ab = jax.lax.dot(s, g, precision=_PREC,
                         preferred_element_type=jnp.float32)
        o_ref[...] = x * ab[:, :_DIM] + ab[:, 128:128 + _DIM]


def kernel(x, weight, bias):
    n = x.shape[0]
    nb = pl.cdiv(n, _BLK)
    vz, vx, ev, m25, m26, gc = _const_mats()
    w2 = weight.reshape(1, _N_SCALAR)
    b2 = bias.reshape(1, _N_SCALAR)

    out = pl.pallas_call(
        functools.partial(_body, n_rows=n),
        grid=(2, nb),
        in_specs=[
            pl.BlockSpec((_BLK, _DIM), lambda p, i: (i, 0)),
            pl.BlockSpec((1, _N_SCALAR), lambda p, i: (0, 0)),
            pl.BlockSpec((1, _N_SCALAR), lambda p, i: (0, 0)),
            pl.BlockSpec((_DIM, 128), lambda p, i: (0, 0)),
            pl.BlockSpec((_DIM, 128), lambda p, i: (0, 0)),
            pl.BlockSpec((128, 256), lambda p, i: (0, 0)),
            pl.BlockSpec((1, 128), lambda p, i: (0, 0)),
            pl.BlockSpec((1, 128), lambda p, i: (0, 0)),
            pl.BlockSpec((1, 128), lambda p, i: (0, 0)),
        ],
        out_specs=pl.BlockSpec((_BLK, _DIM), lambda p, i: (p * i, 0)),
        out_shape=jax.ShapeDtypeStruct((n, _DIM), jnp.float32),
        scratch_shapes=[pltpu.VMEM((1, 128), jnp.float32)],
        compiler_params=pltpu.CompilerParams(
            dimension_semantics=("arbitrary", "arbitrary")),
    )(x, w2, b2, vz, vx, gc, ev, m25, m26)
    return out


# speedup vs baseline: 2.0939x; 1.0062x over previous
import functools

import jax
import jax.numpy as jnp
import numpy as np
from jax.experimental import pallas as pl
from jax.experimental.pallas import tpu as pltpu

_N_SCALAR = 32
_N_VEC = 16
_N_TEN = 8
_DIM = _N_SCALAR + 3 * _N_VEC + 5 * _N_TEN
_NG = _N_VEC + _N_TEN
_EPS = 1e-05
_BLK = 16384
_PREC = jax.lax.Precision.DEFAULT


def _group_cols():
    pairs = []
    for g in range(_N_VEC):
        for k in range(3):
            pairs.append((_N_SCALAR + 3 * g + k, g))
    for t in range(_N_TEN):
        for k in range(5):
            pairs.append((_N_SCALAR + 3 * _N_VEC + 5 * t + k, _N_VEC + t))
    return pairs


def _const_mats():
    vz = np.zeros((_DIM, 128), np.float32)
    for c, g in _group_cols():
        vz[c, g] = 1.0
    vz[:_N_SCALAR, 24] = 1.0 / _N_SCALAR
    vz[:_N_SCALAR, 25] = 1.0 / _N_SCALAR
    vx = np.zeros((_DIM, 128), np.float32)
    vx[:_N_SCALAR, 24] = 1.0 / _N_SCALAR
    vx[:_N_SCALAR, 25] = 1.0 / _N_SCALAR
    ev = np.zeros((1, 128), np.float32)
    ev[0, 24] = _EPS
    ev[0, 25] = _EPS
    m25 = np.zeros((1, 128), np.float32)
    m25[0, 25] = 1.0
    m26 = np.zeros((1, 128), np.float32)
    m26[0, 26] = 1.0
    gc = np.zeros((128, 256), np.float32)
    for c, g in _group_cols():
        gc[g, c] = 1.0
    return (jnp.asarray(vz), jnp.asarray(vx), jnp.asarray(ev),
            jnp.asarray(m25), jnp.asarray(m26), jnp.asarray(gc))


def _body(x_ref, w_ref, b_ref, vz_ref, vx_ref, gc_ref, ev_ref, m25_ref,
          m26_ref, o_ref, acc_ref, *, n_rows):
    p = pl.program_id(0)
    i = pl.program_id(1)
    x = x_ref[...]

    @pl.when(p == 0)
    def _phase0():
        norm2 = jax.lax.dot(x * x, vz_ref[...], precision=_PREC,
                            preferred_element_type=jnp.float32)
        n2c = jnp.maximum(norm2, 1e-12)
        norm = jnp.maximum(norm2 * jax.lax.rsqrt(n2c), 1e-06)
        rows = jax.lax.broadcasted_iota(jnp.int32, norm.shape, 0) + i * _BLK
        norm = jnp.where(rows < n_rows, norm, 0.0)
        psum = jnp.sum(norm, axis=0, keepdims=True)

        @pl.when(i == 0)
        def _():
            acc_ref[...] = jnp.zeros_like(acc_ref)

        acc_ref[...] += psum

    @pl.when(p == 1)
    def _phase1():
        mz = jax.lax.dot(x * x, vz_ref[...], precision=_PREC,
                         preferred_element_type=jnp.float32)
        mx = jax.lax.dot(x, vx_ref[...], precision=_PREC,
                         preferred_element_type=jnp.float32)
        t = jnp.maximum(mz - mx * mx + ev_ref[...], 1e-12)
        rall = jax.lax.rsqrt(t)
        col1 = jax.lax.broadcasted_iota(jnp.int32, (1, 128), 1)
        coef = jnp.where(col1 < _NG, acc_ref[...] * (1.0 / n_rows),
                         jnp.where(col1 == _NG, 1.0, 0.0))
        s = rall * coef + (mx * rall) * m25_ref[...] + m26_ref[...]
        row = jax.lax.broadcasted_iota(jnp.int32, (128, 256), 0)
        col = jax.lax.broadcasted_iota(jnp.int32, (128, 256), 1)
        zpad = jnp.zeros((1, 128 - _N_SCALAR), jnp.float32)
        w = jnp.concatenate([w_ref[...], zpad], axis=1)
        b = jnp.concatenate([b_ref[...], zpad], axis=1)
        wtile = jnp.concatenate([w, w], axis=1)
        btile = jnp.concatenate([b, b], axis=1)
        in_a = col < 128
        g = gc_ref[...]
        g = jnp.where((row == 24) & in_a, wtile, g)
        g = jnp.where((row == 25) & ~in_a, -wtile, g)
        g = jnp.where((row == 26) & ~in_a, btile, g)
        ab = jax.lax.dot(s, g, precision=_PREC,
                         preferred_element_type=jnp.float32)
        o_ref[...] = x * ab[:, :_DIM] + ab[:, 128:128 + _DIM]


def kernel(x, weight, bias):
    n = x.shape[0]
    nb = pl.cdiv(n, _BLK)
    vz, vx, ev, m25, m26, gc = _const_mats()
    w2 = weight.reshape(1, _N_SCALAR)
    b2 = bias.reshape(1, _N_SCALAR)

    out = pl.pallas_call(
        functools.partial(_body, n_rows=n),
        grid=(2, nb),
        in_specs=[
            pl.BlockSpec((_BLK, _DIM), lambda p, i: (i, 0)),
            pl.BlockSpec((1, _N_SCALAR), lambda p, i: (0, 0)),
            pl.BlockSpec((1, _N_SCALAR), lambda p, i: (0, 0)),
            pl.BlockSpec((_DIM, 128), lambda p, i: (0, 0)),
            pl.BlockSpec((_DIM, 128), lambda p, i: (0, 0)),
            pl.BlockSpec((128, 256), lambda p, i: (0, 0)),
            pl.BlockSpec((1, 128), lambda p, i: (0, 0)),
            pl.BlockSpec((1, 128), lambda p, i: (0, 0)),
            pl.BlockSpec((1, 128), lambda p, i: (0, 0)),
        ],
        out_specs=pl.BlockSpec((_BLK, _DIM), lambda p, i: (p * i, 0)),
        out_shape=jax.ShapeDtypeStruct((n, _DIM), jnp.float32),
        scratch_shapes=[pltpu.VMEM((1, 128), jnp.float32)],
        compiler_params=pltpu.CompilerParams(
            dimension_semantics=("arbitrary", "arbitrary")),
    )(x, w2, b2, vz, vx, gc, ev, m25, m26)
    return out


# speedup vs baseline: 4.1874x; 1.9998x over previous
import functools

import jax
import jax.numpy as jnp
import numpy as np
from jax.experimental import pallas as pl
from jax.experimental.pallas import tpu as pltpu

_N_SCALAR = 32
_N_VEC = 16
_N_TEN = 8
_DIM = _N_SCALAR + 3 * _N_VEC + 5 * _N_TEN
_NG = _N_VEC + _N_TEN
_EPS = 1e-05
_CB = 8192
_PREC = jax.lax.Precision.DEFAULT


def _group_cols():
    pairs = []
    for g in range(_N_VEC):
        for k in range(3):
            pairs.append((_N_SCALAR + 3 * g + k, g))
    for t in range(_N_TEN):
        for k in range(5):
            pairs.append((_N_SCALAR + 3 * _N_VEC + 5 * t + k, _N_VEC + t))
    return pairs


def _const_mats():
    vzt = np.zeros((128, _DIM), np.float32)
    for c, g in _group_cols():
        vzt[g, c] = 1.0
    vzt[24, :_N_SCALAR] = 1.0 / _N_SCALAR
    vzt[25, :_N_SCALAR] = 1.0 / _N_SCALAR
    vxt = np.zeros((128, _DIM), np.float32)
    vxt[24, :_N_SCALAR] = 1.0 / _N_SCALAR
    vxt[25, :_N_SCALAR] = 1.0 / _N_SCALAR
    ev = np.zeros((128, 1), np.float32)
    ev[24, 0] = _EPS
    ev[25, 0] = _EPS
    m25 = np.zeros((128, 1), np.float32)
    m25[25, 0] = 1.0
    m26 = np.zeros((128, 1), np.float32)
    m26[26, 0] = 1.0
    gct = np.zeros((256, 128), np.float32)
    for c, g in _group_cols():
        gct[c, g] = 1.0
    return (jnp.asarray(vzt), jnp.asarray(vxt), jnp.asarray(ev),
            jnp.asarray(m25), jnp.asarray(m26), jnp.asarray(gct))


def _body(x_ref, w_ref, b_ref, vzt_ref, vxt_ref, gct_ref, ev_ref, m25_ref,
          m26_ref, o_ref, acc_ref, *, n_rows):
    p = pl.program_id(0)
    i = pl.program_id(1)
    x = x_ref[...]

    @pl.when(p == 0)
    def _phase0():
        norm2 = jax.lax.dot(vzt_ref[...], x * x, precision=_PREC,
                            preferred_element_type=jnp.float32)
        n2c = jnp.maximum(norm2, 1e-12)
        norm = jnp.maximum(norm2 * jax.lax.rsqrt(n2c), 1e-06)
        cols = jax.lax.broadcasted_iota(jnp.int32, (1, _CB), 1) + i * _CB
        norm = jnp.where(cols < n_rows, norm, 0.0)
        psum = jnp.sum(norm, axis=1, keepdims=True)

        @pl.when(i == 0)
        def _():
            acc_ref[...] = jnp.zeros_like(acc_ref)

        acc_ref[...] += psum

    @pl.when(p == 1)
    def _phase1():
        mz = jax.lax.dot(vzt_ref[...], x * x, precision=_PREC,
                         preferred_element_type=jnp.float32)
        mx = jax.lax.dot(vxt_ref[...], x, precision=_PREC,
                         preferred_element_type=jnp.float32)
        t = jnp.maximum(mz - mx * mx + ev_ref[...], 1e-12)
        rall = jax.lax.rsqrt(t)
        row1 = jax.lax.broadcasted_iota(jnp.int32, (128, 1), 0)
        coef = jnp.where(row1 < _NG, acc_ref[...] * (1.0 / n_rows),
                         jnp.where(row1 == _NG, 1.0, 0.0))
        s = rall * coef + (mx * rall) * m25_ref[...] + m26_ref[...]
        row = jax.lax.broadcasted_iota(jnp.int32, (256, 128), 0)
        col = jax.lax.broadcasted_iota(jnp.int32, (256, 128), 1)
        zpad = jnp.zeros((128 - _N_SCALAR, 1), jnp.float32)
        w128 = jnp.concatenate([w_ref[...], zpad], axis=0)
        b128 = jnp.concatenate([b_ref[...], zpad], axis=0)
        w256 = jnp.concatenate([w128, w128], axis=0)
        b256 = jnp.concatenate([b128, b128], axis=0)
        in_a = row < 128
        g = gct_ref[...]
        g = jnp.where((col == 24) & in_a, w256, g)
        g = jnp.where((col == 25) & ~in_a, -w256, g)
        g = jnp.where((col == 26) & ~in_a, b256, g)
        ab = jax.lax.dot(g, s, precision=_PREC,
                         preferred_element_type=jnp.float32)
        o_ref[...] = x * ab[:_DIM, :] + ab[128:128 + _DIM, :]


def kernel(x, weight, bias):
    n = x.shape[0]
    nb = pl.cdiv(n, _CB)
    vzt, vxt, ev, m25, m26, gct = _const_mats()
    xt = x.T
    w2 = weight.reshape(_N_SCALAR, 1)
    b2 = bias.reshape(_N_SCALAR, 1)

    out_t = pl.pallas_call(
        functools.partial(_body, n_rows=n),
        grid=(2, nb),
        in_specs=[
            pl.BlockSpec((_DIM, _CB), lambda p, i: (0, i)),
            pl.BlockSpec((_N_SCALAR, 1), lambda p, i: (0, 0)),
            pl.BlockSpec((_N_SCALAR, 1), lambda p, i: (0, 0)),
            pl.BlockSpec((128, _DIM), lambda p, i: (0, 0)),
            pl.BlockSpec((128, _DIM), lambda p, i: (0, 0)),
            pl.BlockSpec((256, 128), lambda p, i: (0, 0)),
            pl.BlockSpec((128, 1), lambda p, i: (0, 0)),
            pl.BlockSpec((128, 1), lambda p, i: (0, 0)),
            pl.BlockSpec((128, 1), lambda p, i: (0, 0)),
        ],
        out_specs=pl.BlockSpec((_DIM, _CB), lambda p, i: (0, p * i)),
        out_shape=jax.ShapeDtypeStruct((_DIM, n), jnp.float32),
        scratch_shapes=[pltpu.VMEM((128, 1), jnp.float32)],
        compiler_params=pltpu.CompilerParams(
            dimension_semantics=("arbitrary", "arbitrary")),
    )(xt, w2, b2, vzt, vxt, gct, ev, m25, m26)
    return out_t.T
